# baseline (speedup 1.0000x reference)
import functools
import numpy as np
import jax
import jax.numpy as jnp
from jax import lax
from jax.experimental import pallas as pl
from jax.experimental.pallas import tpu as pltpu
from jax.experimental.pallas import tpu_sc as plsc

_NUM_MASKS = 2
_MAX_WIDTH = 40
_ALIGN = 128
_WMAX = 512


def _span_list(B, T):
    rng = np.random.RandomState(0)
    spans = []
    for b in range(B):
        for _ in range(_NUM_MASKS):
            width = int(rng.randint(1, _MAX_WIDTH + 1))
            if T - width <= 0:
                continue
            start = int(rng.randint(0, T - width))
            spans.append((b, start, width))
    return spans


def _merged_windows(B, T):
    spans = _span_list(B, T)
    per_b = {}
    for b, s, w in spans:
        a0 = (s // _ALIGN) * _ALIGN
        a1 = min(T, -(-(s + w) // _ALIGN) * _ALIGN)
        per_b.setdefault(b, []).append((a0, a1))
    windows = []
    for b in sorted(per_b):
        ivs = sorted(per_b[b])
        merged = [list(ivs[0])]
        for a0, a1 in ivs[1:]:
            if a0 <= merged[-1][1]:
                merged[-1][1] = max(merged[-1][1], a1)
            else:
                merged.append([a0, a1])
        for a0, a1 in merged:
            local = [(s - a0, w) for bb, s, w in spans
                     if bb == b and s >= a0 and s + w <= a1]
            windows.append((b, a0, a1 - a0, local))
    return windows


def kernel(x):
    B, C, T = x.shape
    windows = _merged_windows(B, T)
    mesh = plsc.VectorSubcoreMesh(
        core_axis_name="c", subcore_axis_name="s", num_cores=2,
        num_subcores=16)
    Cb = C // 4

    @functools.partial(
        pl.kernel,
        out_type=jax.ShapeDtypeStruct((B, C, T), x.dtype),
        mesh=mesh,
        scratch_types=[pltpu.VMEM((Cb, _WMAX), x.dtype)],
        compiler_params=pltpu.CompilerParams(use_tc_tiling_on_sc=False),
    )
    def sc_fn(x_hbm, out_hbm, wbuf):
        wid = lax.axis_index("c") * 16 + lax.axis_index("s")
        b = wid // 4
        c0 = (wid % 4) * Cb
        pltpu.sync_copy(
            x_hbm.at[b, pl.ds(c0, Cb), :],
            out_hbm.at[b, pl.ds(c0, Cb), :],
        )
        zvec = jnp.zeros((16,), x.dtype)
        lane = lax.iota(jnp.int32, 16)
        for wb, a0, wlen, local in windows:
            @pl.when(b == wb)
            def _(wb=wb, a0=a0, wlen=wlen, local=local):
                pltpu.sync_copy(
                    x_hbm.at[wb, pl.ds(c0, Cb), pl.ds(a0, wlen)],
                    wbuf.at[:, pl.ds(0, wlen)],
                )

                def row_body(r, carry):
                    for off, w in local:
                        k_lo = off // 16
                        k_hi = (off + w - 1) // 16
                        for k in range(k_lo, k_hi + 1):
                            if 16 * k >= off and 16 * k + 16 <= off + w:
                                wbuf[r, pl.ds(16 * k, 16)] = zvec
                            else:
                                v = wbuf[r, pl.ds(16 * k, 16)]
                                t = lane + 16 * k
                                keep = (t < off) | (t >= off + w)
                                wbuf[r, pl.ds(16 * k, 16)] = jnp.where(
                                    keep, v, jnp.float32(0.0))
                    return carry

                lax.fori_loop(0, Cb, row_body, 0)
                pltpu.sync_copy(
                    wbuf.at[:, pl.ds(0, wlen)],
                    out_hbm.at[wb, pl.ds(c0, Cb), pl.ds(a0, wlen)],
                )

    return sc_fn(x)


# speedup vs baseline: 51.8432x; 51.8432x over previous
import numpy as np
import jax
import jax.numpy as jnp
from jax.experimental import pallas as pl
from jax.experimental.pallas import tpu as pltpu

_NUM_MASKS = 2
_MAX_WIDTH = 40


def _span_mask(B, T):
    rng = np.random.RandomState(0)
    mask = np.ones((B, 1, T), dtype=np.float32)
    for b in range(B):
        for _ in range(_NUM_MASKS):
            width = int(rng.randint(1, _MAX_WIDTH + 1))
            if T - width <= 0:
                continue
            start = int(rng.randint(0, T - width))
            mask[b, 0, start:start + width] = 0.0
    return mask


def _mask_mul(x_ref, m_ref, o_ref):
    o_ref[...] = x_ref[...] * m_ref[...]


def kernel(x):
    B, C, T = x.shape
    mask = jnp.asarray(_span_mask(B, T))

    Ct = 256
    grid = (B, C // Ct)
    return pl.pallas_call(
        _mask_mul,
        grid=grid,
        in_specs=[
            pl.BlockSpec((1, Ct, T), lambda b, c: (b, c, 0)),
            pl.BlockSpec((1, 1, T), lambda b, c: (b, 0, 0)),
        ],
        out_specs=pl.BlockSpec((1, Ct, T), lambda b, c: (b, c, 0)),
        out_shape=jax.ShapeDtypeStruct((B, C, T), x.dtype),
        compiler_params=pltpu.CompilerParams(vmem_limit_bytes=100 * 1024 * 1024),
    )(x, mask)


# speedup vs baseline: 51.8790x; 1.0007x over previous
import numpy as np
import jax
import jax.numpy as jnp
from jax.experimental import pallas as pl
from jax.experimental.pallas import tpu as pltpu

_NUM_MASKS = 2
_MAX_WIDTH = 40


def _span_mask(B, T):
    rng = np.random.RandomState(0)
    mask = np.ones((B, 1, T), dtype=np.float32)
    for b in range(B):
        for _ in range(_NUM_MASKS):
            width = int(rng.randint(1, _MAX_WIDTH + 1))
            if T - width <= 0:
                continue
            start = int(rng.randint(0, T - width))
            mask[b, 0, start:start + width] = 0.0
    return mask


def _mask_mul(x_ref, m_ref, o_ref):
    o_ref[...] = x_ref[...] * m_ref[...]


def kernel(x):
    B, C, T = x.shape
    mask = jnp.asarray(_span_mask(B, T))

    Ct = 256
    grid = (B, C // Ct)
    return pl.pallas_call(
        _mask_mul,
        grid=grid,
        in_specs=[
            pl.BlockSpec((1, Ct, T), lambda b, c: (b, c, 0)),
            pl.BlockSpec((1, 1, T), lambda b, c: (b, 0, 0)),
        ],
        out_specs=pl.BlockSpec((1, Ct, T), lambda b, c: (b, c, 0)),
        out_shape=jax.ShapeDtypeStruct((B, C, T), x.dtype),
        compiler_params=pltpu.CompilerParams(vmem_limit_bytes=100 * 1024 * 1024, dimension_semantics=("parallel", "parallel")),
    )(x, mask)
